# baseline (speedup 1.0000x reference)
import jax
import jax.numpy as jnp
from jax import lax
from jax.experimental import pallas as pl
from jax.experimental.pallas import tpu as pltpu
from jax.experimental.pallas import tpu_sc as plsc

NUM_USERS = 100000
NUM_ITEMS = 1000000
D = 32
H = 16
B = 16384
NSUB = 16
NCORE = 2
NPT = B // NSUB
NCHUNK = NPT // 128
NGRP = NPT // 16


def _sc_gather_body(user_table, item_table, item_pop, u_idx, i_idx, j_idx,
                    p_i_f, p_i_s, p_j_f, p_j_s, rel_out, d_i_out, d_j_out,
                    d_u_out, reg_part,
                    uidx_v, idx_v, idx2_v,
                    urows_v, irows_v,
                    pf_v, ps_v, d_itm_v, d_usr_v,
                    popi_v, popj_v, rel_v, reg_buf,
                    sem_u, sem_i):
  core = lax.axis_index("c")
  sub = lax.axis_index("s")
  wid = core * NSUB + sub
  base = sub * NPT
  c0 = core == 0
  iota = lax.iota(jnp.int32, 16)

  for k in range(NCHUNK):
    pltpu.sync_copy(u_idx.at[pl.ds(base + k * 128, 128)], uidx_v.at[k])

  @pl.when(c0)
  def _():
    for k in range(NCHUNK):
      pltpu.sync_copy(i_idx.at[pl.ds(base + k * 128, 128)], idx_v.at[k])

  @pl.when(jnp.logical_not(c0))
  def _():
    for k in range(NCHUNK):
      pltpu.sync_copy(j_idx.at[pl.ds(base + k * 128, 128)], idx_v.at[k])
      pltpu.sync_copy(i_idx.at[pl.ds(base + k * 128, 128)], idx2_v.at[k])

  copies = []
  for k in range(NCHUNK):
    copies.append(pltpu.async_copy(
        user_table.at[uidx_v.at[k]], urows_v.at[pl.ds(k * 128, 128)], sem_u))
    copies.append(pltpu.async_copy(
        item_table.at[idx_v.at[k]], irows_v.at[pl.ds(k * 128, 128)], sem_i))

  @pl.when(jnp.logical_not(c0))
  def _():
    for k in range(NCHUNK):
      pltpu.sync_copy(item_pop.at[idx2_v.at[k]],
                      popi_v.at[pl.ds(k * 128, 128)])
      pltpu.sync_copy(item_pop.at[idx_v.at[k]],
                      popj_v.at[pl.ds(k * 128, 128)])

  for c in copies:
    c.wait()

  zero16 = jnp.zeros((16,), jnp.float32)

  def _grp(g, carry):
    usq, isq = carry
    r0 = g * 16
    rvec = r0 + iota
    pf = zero16
    ps = zero16
    du = zero16
    di = zero16
    for k in range(D):
      ck = jnp.full((16,), k, jnp.int32)
      uc = plsc.load_gather(urows_v, [rvec, ck])
      ic = plsc.load_gather(irows_v, [rvec, ck])
      usq = usq + uc * uc
      isq = isq + ic * ic
      if k < H:
        pf = pf + uc * ic
        du = du + uc
        di = di + ic
      else:
        ps = ps + uc * ic
        du = du - uc
        di = di - ic
    sl = pl.ds(r0, 16)
    pf_v[sl] = pf
    ps_v[sl] = ps
    d_itm_v[sl] = di
    d_usr_v[sl] = du
    rel_v[sl] = jnp.where(popi_v[sl] > popj_v[sl], 1.0, 0.0)
    return usq, isq

  usq, isq = lax.fori_loop(0, NGRP, _grp, (zero16, zero16))

  c0f = jnp.where(c0, 1.0, 0.0).astype(jnp.float32)
  reg_buf[...] = isq + c0f * usq
  pltpu.sync_copy(reg_buf, reg_part.at[wid])

  out_sl = pl.ds(base, NPT)

  @pl.when(c0)
  def _():
    pltpu.sync_copy(pf_v, p_i_f.at[out_sl])
    pltpu.sync_copy(ps_v, p_i_s.at[out_sl])
    pltpu.sync_copy(d_itm_v, d_i_out.at[out_sl])
    pltpu.sync_copy(d_usr_v, d_u_out.at[out_sl])

  @pl.when(jnp.logical_not(c0))
  def _():
    pltpu.sync_copy(pf_v, p_j_f.at[out_sl])
    pltpu.sync_copy(ps_v, p_j_s.at[out_sl])
    pltpu.sync_copy(d_itm_v, d_j_out.at[out_sl])
    pltpu.sync_copy(rel_v, rel_out.at[out_sl])


def _sc_gather_stage(user_table, item_table, item_pop, u_idx, i_idx, j_idx):
  f32 = jnp.float32
  i32 = jnp.int32
  mesh = plsc.VectorSubcoreMesh(core_axis_name="c", subcore_axis_name="s")
  out_type = (
      jax.ShapeDtypeStruct((B,), f32),
      jax.ShapeDtypeStruct((B,), f32),
      jax.ShapeDtypeStruct((B,), f32),
      jax.ShapeDtypeStruct((B,), f32),
      jax.ShapeDtypeStruct((B,), f32),
      jax.ShapeDtypeStruct((B,), f32),
      jax.ShapeDtypeStruct((B,), f32),
      jax.ShapeDtypeStruct((B,), f32),
      jax.ShapeDtypeStruct((NCORE * NSUB, 16), f32),
  )
  scratch = [
      pltpu.VMEM((NCHUNK, 128), i32),
      pltpu.VMEM((NCHUNK, 128), i32),
      pltpu.VMEM((NCHUNK, 128), i32),
      pltpu.VMEM((NPT, D), f32),
      pltpu.VMEM((NPT, D), f32),
      pltpu.VMEM((NPT,), f32),
      pltpu.VMEM((NPT,), f32),
      pltpu.VMEM((NPT,), f32),
      pltpu.VMEM((NPT,), f32),
      pltpu.VMEM((NPT,), f32),
      pltpu.VMEM((NPT,), f32),
      pltpu.VMEM((NPT,), f32),
      pltpu.VMEM((16,), f32),
      pltpu.SemaphoreType.DMA,
      pltpu.SemaphoreType.DMA,
  ]
  fn = pl.kernel(_sc_gather_body, out_type=out_type, mesh=mesh,
                 scratch_types=scratch,
                 compiler_params=pltpu.CompilerParams(
                     needs_layout_passes=False,
                     use_tc_tiling_on_sc=False))
  return fn(user_table, item_table, item_pop, u_idx, i_idx, j_idx)


def _sc_dedupe_body(u_idx, i_idx, j_idx, d_i, d_j, d_u,
                    disc_part,
                    idx_v, idxu_v, pos_v, win_v, winu_v, d_v, du_v,
                    disc_buf, claim_itm, claim_usr):
  core = lax.axis_index("c")
  sub = lax.axis_index("s")
  wid = core * NSUB + sub
  base = sub * NPT
  c0 = core == 0
  iota = lax.iota(jnp.int32, 16)

  @pl.when(c0)
  def _():
    for k in range(NCHUNK):
      pltpu.sync_copy(i_idx.at[pl.ds(base + k * 128, 128)], idx_v.at[k])
      pltpu.sync_copy(u_idx.at[pl.ds(base + k * 128, 128)], idxu_v.at[k])
    pltpu.sync_copy(d_i.at[pl.ds(base, NPT)], d_v)
    pltpu.sync_copy(d_u.at[pl.ds(base, NPT)], du_v)

  @pl.when(jnp.logical_not(c0))
  def _():
    for k in range(NCHUNK):
      pltpu.sync_copy(j_idx.at[pl.ds(base + k * 128, 128)], idx_v.at[k])
    pltpu.sync_copy(d_j.at[pl.ds(base, NPT)], d_v)

  def _posb(g, carry):
    pos_v[pl.ds(g * 16, 16)] = base + g * 16 + iota
    return carry
  lax.fori_loop(0, NGRP, _posb, 0)

  for k in range(NCHUNK):
    pltpu.sync_copy(pos_v.at[pl.ds(k * 128, 128)], claim_itm.at[idx_v.at[k]])

  @pl.when(c0)
  def _():
    for k in range(NCHUNK):
      pltpu.sync_copy(pos_v.at[pl.ds(k * 128, 128)],
                      claim_usr.at[idxu_v.at[k]])

  plsc.subcore_barrier()

  for k in range(NCHUNK):
    pltpu.sync_copy(claim_itm.at[idx_v.at[k]], win_v.at[pl.ds(k * 128, 128)])

  @pl.when(c0)
  def _():
    for k in range(NCHUNK):
      pltpu.sync_copy(claim_usr.at[idxu_v.at[k]],
                      winu_v.at[pl.ds(k * 128, 128)])

  c0f = jnp.where(c0, 1.0, 0.0).astype(jnp.float32)
  zero16 = jnp.zeros((16,), jnp.float32)

  def _msum(g, acc):
    sl = pl.ds(g * 16, 16)
    p = pos_v[sl]
    acc = (acc + jnp.where(win_v[sl] == p, d_v[sl], 0.0)
           + c0f * jnp.where(winu_v[sl] == p, du_v[sl], 0.0))
    return acc

  disc_buf[...] = lax.fori_loop(0, NGRP, _msum, zero16)
  pltpu.sync_copy(disc_buf, disc_part.at[wid])


def _sc_dedupe_stage(u_idx, i_idx, j_idx, d_i, d_j, d_u):
  f32 = jnp.float32
  i32 = jnp.int32
  mesh = plsc.VectorSubcoreMesh(core_axis_name="c", subcore_axis_name="s")
  out_type = jax.ShapeDtypeStruct((NCORE * NSUB, 16), f32)
  scratch = [
      pltpu.VMEM((NCHUNK, 128), i32),
      pltpu.VMEM((NCHUNK, 128), i32),
      pltpu.VMEM((NPT,), i32),
      pltpu.VMEM((NPT,), i32),
      pltpu.VMEM((NPT,), i32),
      pltpu.VMEM((NPT,), f32),
      pltpu.VMEM((NPT,), f32),
      pltpu.VMEM((16,), f32),
      pltpu.VMEM_SHARED((NUM_ITEMS,), i32),
      pltpu.VMEM_SHARED((NUM_USERS,), i32),
  ]
  fn = pl.kernel(_sc_dedupe_body, out_type=out_type, mesh=mesh,
                 scratch_types=scratch,
                 compiler_params=pltpu.CompilerParams(
                     needs_layout_passes=False,
                     use_tc_tiling_on_sc=False))
  return fn(u_idx, i_idx, j_idx, d_i, d_j, d_u)


def _tc_body(pif, pis, pjf, pjs, rel, regp, discp,
             o_click, o_int, o_p1, o_p2, o_disc, o_reg):
  def logsig(x):
    return jnp.minimum(x, 0.0) - jnp.log1p(jnp.exp(-jnp.abs(x)))

  a_pif = pif[...]
  a_pis = pis[...]
  a_pjf = pjf[...]
  a_pjs = pjs[...]
  relb = rel[...] > 0.5
  xf = (a_pif + a_pis) - (a_pjf + a_pjs)
  o_click[0, 0] = -jnp.sum(logsig(xf))
  o_int[0, 0] = -jnp.sum(jnp.where(relb, logsig(a_pif - a_pjf), 0.0))
  o_p1[0, 0] = -jnp.sum(jnp.where(relb, logsig(a_pjs - a_pis), 0.0))
  o_p2[0, 0] = -jnp.sum(jnp.where(~relb, logsig(a_pis - a_pjs), 0.0))
  o_disc[0, 0] = -jnp.sum(discp[...])
  o_reg[0, 0] = 0.5 * jnp.sum(regp[...]) / float(B)


def kernel(user_table, item_table, item_popularity, user_indices,
           item_i_indices, item_j_indices):
  f32 = jnp.float32
  u_idx = user_indices.astype(jnp.int32)
  i_idx = item_i_indices.astype(jnp.int32)
  j_idx = item_j_indices.astype(jnp.int32)
  (pif, pis, pjf, pjs, rel, d_i, d_j, d_u, regp) = _sc_gather_stage(
      user_table, item_table, item_popularity, u_idx, i_idx, j_idx)
  discp = _sc_dedupe_stage(u_idx, i_idx, j_idx, d_i, d_j, d_u)
  sq = lambda a: a.reshape(128, 128)
  outs = pl.pallas_call(
      _tc_body,
      out_shape=[jax.ShapeDtypeStruct((1, 1), f32)] * 6,
      out_specs=[pl.BlockSpec(memory_space=pltpu.SMEM)] * 6,
  )(sq(pif), sq(pis), sq(pjf), sq(pjs), sq(rel),
    regp.reshape(4, 128), discp.reshape(4, 128))
  click, l_int, l_p1, l_p2, l_disc, l_reg = [o[0, 0] for o in outs]
  return (click, l_int, l_p1, l_p2, l_disc, l_reg)


# speedup vs baseline: 1.0610x; 1.0610x over previous
import jax
import jax.numpy as jnp
from jax import lax
from jax.experimental import pallas as pl
from jax.experimental.pallas import tpu as pltpu
from jax.experimental.pallas import tpu_sc as plsc

NUM_USERS = 100000
NUM_ITEMS = 1000000
D = 32
H = 16
B = 16384
NSUB = 16
NCORE = 2
NPT = B // NSUB
NCHUNK = NPT // 128
NGRP = NPT // 16


def _sc_dedupe_body(u_idx, i_idx, j_idx,
                    mask_i, mask_j, mask_u,
                    idx_v, idxu_v, pos_v, win_v, winu_v, m_v, mu_v,
                    claim_itm, claim_usr, sem):
  core = lax.axis_index("c")
  sub = lax.axis_index("s")
  base = sub * NPT
  c0 = core == 0
  iota = lax.iota(jnp.int32, 16)

  @pl.when(c0)
  def _():
    for k in range(NCHUNK):
      pltpu.sync_copy(i_idx.at[pl.ds(base + k * 128, 128)], idx_v.at[k])
      pltpu.sync_copy(u_idx.at[pl.ds(base + k * 128, 128)], idxu_v.at[k])

  @pl.when(jnp.logical_not(c0))
  def _():
    for k in range(NCHUNK):
      pltpu.sync_copy(j_idx.at[pl.ds(base + k * 128, 128)], idx_v.at[k])

  def _posb(g, carry):
    pos_v[pl.ds(g * 16, 16)] = base + g * 16 + iota
    return carry
  lax.fori_loop(0, NGRP, _posb, 0)

  claims = []
  for k in range(NCHUNK):
    claims.append(pltpu.async_copy(
        pos_v.at[pl.ds(k * 128, 128)], claim_itm.at[idx_v.at[k]], sem))

  @pl.when(c0)
  def _():
    cs = []
    for k in range(NCHUNK):
      cs.append(pltpu.async_copy(
          pos_v.at[pl.ds(k * 128, 128)], claim_usr.at[idxu_v.at[k]], sem))
    for c in cs:
      c.wait()

  for c in claims:
    c.wait()

  plsc.subcore_barrier()

  gathers = []
  for k in range(NCHUNK):
    gathers.append(pltpu.async_copy(
        claim_itm.at[idx_v.at[k]], win_v.at[pl.ds(k * 128, 128)], sem))

  @pl.when(c0)
  def _():
    cs = []
    for k in range(NCHUNK):
      cs.append(pltpu.async_copy(
          claim_usr.at[idxu_v.at[k]], winu_v.at[pl.ds(k * 128, 128)], sem))
    for c in cs:
      c.wait()

  for c in gathers:
    c.wait()

  def _mk(g, carry):
    sl = pl.ds(g * 16, 16)
    p = pos_v[sl]
    m_v[sl] = jnp.where(win_v[sl] == p, 1.0, 0.0)
    mu_v[sl] = jnp.where(winu_v[sl] == p, 1.0, 0.0)
    return carry
  lax.fori_loop(0, NGRP, _mk, 0)

  out_sl = pl.ds(base, NPT)

  @pl.when(c0)
  def _():
    pltpu.sync_copy(m_v, mask_i.at[out_sl])
    pltpu.sync_copy(mu_v, mask_u.at[out_sl])

  @pl.when(jnp.logical_not(c0))
  def _():
    pltpu.sync_copy(m_v, mask_j.at[out_sl])


def _sc_dedupe_stage(u_idx, i_idx, j_idx):
  f32 = jnp.float32
  i32 = jnp.int32
  mesh = plsc.VectorSubcoreMesh(core_axis_name="c", subcore_axis_name="s")
  out_type = (
      jax.ShapeDtypeStruct((B,), f32),
      jax.ShapeDtypeStruct((B,), f32),
      jax.ShapeDtypeStruct((B,), f32),
  )
  scratch = [
      pltpu.VMEM((NCHUNK, 128), i32),
      pltpu.VMEM((NCHUNK, 128), i32),
      pltpu.VMEM((NPT,), i32),
      pltpu.VMEM((NPT,), i32),
      pltpu.VMEM((NPT,), i32),
      pltpu.VMEM((NPT,), f32),
      pltpu.VMEM((NPT,), f32),
      pltpu.VMEM_SHARED((NUM_ITEMS,), i32),
      pltpu.VMEM_SHARED((NUM_USERS,), i32),
      pltpu.SemaphoreType.DMA,
  ]
  fn = pl.kernel(_sc_dedupe_body, out_type=out_type, mesh=mesh,
                 scratch_types=scratch,
                 compiler_params=pltpu.CompilerParams(
                     needs_layout_passes=False,
                     use_tc_tiling_on_sc=False))
  return fn(u_idx, i_idx, j_idx)


def _sc_gather_body(user_table, item_table, item_pop, u_idx, i_idx, j_idx,
                    mask_i, mask_j, mask_u,
                    p_i_f, p_i_s, p_j_f, p_j_s, rel_out, reg_part, disc_part,
                    uidx_v, idx_v, idx2_v,
                    urows_v, irows_v,
                    pf_v, ps_v, popi_v, popj_v, rel_v,
                    m_v, mu_v, part_buf,
                    sem_u, sem_i, sem_p):
  core = lax.axis_index("c")
  sub = lax.axis_index("s")
  wid = core * NSUB + sub
  base = sub * NPT
  c0 = core == 0
  iota = lax.iota(jnp.int32, 16)
  in_sl = pl.ds(base, NPT)

  pltpu.sync_copy(u_idx.at[in_sl], uidx_v)
  mcp = [pltpu.async_copy(mask_u.at[in_sl], mu_v, sem_p)]

  @pl.when(c0)
  def _():
    pltpu.sync_copy(i_idx.at[in_sl], idx_v)
    pltpu.async_copy(mask_i.at[in_sl], m_v, sem_p).wait()

  @pl.when(jnp.logical_not(c0))
  def _():
    pltpu.sync_copy(j_idx.at[in_sl], idx_v)
    pltpu.sync_copy(i_idx.at[in_sl], idx2_v)
    pltpu.async_copy(mask_j.at[in_sl], m_v, sem_p).wait()

  copies = []
  for k in range(NCHUNK):
    copies.append(pltpu.async_copy(
        user_table.at[uidx_v.at[pl.ds(k * 128, 128)]],
        urows_v.at[pl.ds(k * 128, 128)], sem_u))
    copies.append(pltpu.async_copy(
        item_table.at[idx_v.at[pl.ds(k * 128, 128)]],
        irows_v.at[pl.ds(k * 128, 128)], sem_i))

  @pl.when(jnp.logical_not(c0))
  def _():
    cs = []
    for k in range(NCHUNK):
      cs.append(pltpu.async_copy(
          item_pop.at[idx2_v.at[pl.ds(k * 128, 128)]],
          popi_v.at[pl.ds(k * 128, 128)], sem_p))
      cs.append(pltpu.async_copy(
          item_pop.at[idx_v.at[pl.ds(k * 128, 128)]],
          popj_v.at[pl.ds(k * 128, 128)], sem_p))
    for c in cs:
      c.wait()

  for c in copies:
    c.wait()
  for c in mcp:
    c.wait()

  zero16 = jnp.zeros((16,), jnp.float32)
  c0f = jnp.where(c0, 1.0, 0.0).astype(jnp.float32)

  def _grp(g, carry):
    usq, isq, disc = carry
    r0 = g * 16
    rvec = r0 + iota
    pf = zero16
    ps = zero16
    du = zero16
    di = zero16
    for k in range(D):
      ck = jnp.full((16,), k, jnp.int32)
      uc = plsc.load_gather(urows_v, [rvec, ck])
      ic = plsc.load_gather(irows_v, [rvec, ck])
      usq = usq + uc * uc
      isq = isq + ic * ic
      if k < H:
        pf = pf + uc * ic
        du = du + uc
        di = di + ic
      else:
        ps = ps + uc * ic
        du = du - uc
        di = di - ic
    sl = pl.ds(r0, 16)
    pf_v[sl] = pf
    ps_v[sl] = ps
    rel_v[sl] = jnp.where(popi_v[sl] > popj_v[sl], 1.0, 0.0)
    disc = disc + m_v[sl] * di + c0f * (mu_v[sl] * du)
    return usq, isq, disc

  usq, isq, disc = lax.fori_loop(0, NGRP, _grp, (zero16, zero16, zero16))

  part_buf[0, :] = isq + c0f * usq
  part_buf[1, :] = disc
  pltpu.sync_copy(part_buf.at[0], reg_part.at[wid])
  pltpu.sync_copy(part_buf.at[1], disc_part.at[wid])

  @pl.when(c0)
  def _():
    pltpu.sync_copy(pf_v, p_i_f.at[in_sl])
    pltpu.sync_copy(ps_v, p_i_s.at[in_sl])

  @pl.when(jnp.logical_not(c0))
  def _():
    pltpu.sync_copy(pf_v, p_j_f.at[in_sl])
    pltpu.sync_copy(ps_v, p_j_s.at[in_sl])
    pltpu.sync_copy(rel_v, rel_out.at[in_sl])


def _sc_gather_stage(user_table, item_table, item_pop, u_idx, i_idx, j_idx,
                     mask_i, mask_j, mask_u):
  f32 = jnp.float32
  i32 = jnp.int32
  mesh = plsc.VectorSubcoreMesh(core_axis_name="c", subcore_axis_name="s")
  out_type = (
      jax.ShapeDtypeStruct((B,), f32),
      jax.ShapeDtypeStruct((B,), f32),
      jax.ShapeDtypeStruct((B,), f32),
      jax.ShapeDtypeStruct((B,), f32),
      jax.ShapeDtypeStruct((B,), f32),
      jax.ShapeDtypeStruct((NCORE * NSUB, 16), f32),
      jax.ShapeDtypeStruct((NCORE * NSUB, 16), f32),
  )
  scratch = [
      pltpu.VMEM((NPT,), i32),
      pltpu.VMEM((NPT,), i32),
      pltpu.VMEM((NPT,), i32),
      pltpu.VMEM((NPT, D), f32),
      pltpu.VMEM((NPT, D), f32),
      pltpu.VMEM((NPT,), f32),
      pltpu.VMEM((NPT,), f32),
      pltpu.VMEM((NPT,), f32),
      pltpu.VMEM((NPT,), f32),
      pltpu.VMEM((NPT,), f32),
      pltpu.VMEM((NPT,), f32),
      pltpu.VMEM((NPT,), f32),
      pltpu.VMEM((2, 16), f32),
      pltpu.SemaphoreType.DMA,
      pltpu.SemaphoreType.DMA,
      pltpu.SemaphoreType.DMA,
  ]
  fn = pl.kernel(_sc_gather_body, out_type=out_type, mesh=mesh,
                 scratch_types=scratch,
                 compiler_params=pltpu.CompilerParams(
                     needs_layout_passes=False,
                     use_tc_tiling_on_sc=False))
  return fn(user_table, item_table, item_pop, u_idx, i_idx, j_idx,
            mask_i, mask_j, mask_u)


def _tc_body(pif, pis, pjf, pjs, rel, regp, discp,
             o_click, o_int, o_p1, o_p2, o_disc, o_reg):
  def logsig(x):
    return jnp.minimum(x, 0.0) - jnp.log1p(jnp.exp(-jnp.abs(x)))

  a_pif = pif[...]
  a_pis = pis[...]
  a_pjf = pjf[...]
  a_pjs = pjs[...]
  relb = rel[...] > 0.5
  xf = (a_pif + a_pis) - (a_pjf + a_pjs)
  o_click[0, 0] = -jnp.sum(logsig(xf))
  o_int[0, 0] = -jnp.sum(jnp.where(relb, logsig(a_pif - a_pjf), 0.0))
  o_p1[0, 0] = -jnp.sum(jnp.where(relb, logsig(a_pjs - a_pis), 0.0))
  o_p2[0, 0] = -jnp.sum(jnp.where(~relb, logsig(a_pis - a_pjs), 0.0))
  o_disc[0, 0] = -jnp.sum(discp[...])
  o_reg[0, 0] = 0.5 * jnp.sum(regp[...]) / float(B)


def kernel(user_table, item_table, item_popularity, user_indices,
           item_i_indices, item_j_indices):
  f32 = jnp.float32
  u_idx = user_indices.astype(jnp.int32)
  i_idx = item_i_indices.astype(jnp.int32)
  j_idx = item_j_indices.astype(jnp.int32)
  mask_i, mask_j, mask_u = _sc_dedupe_stage(u_idx, i_idx, j_idx)
  (pif, pis, pjf, pjs, rel, regp, discp) = _sc_gather_stage(
      user_table, item_table, item_popularity, u_idx, i_idx, j_idx,
      mask_i, mask_j, mask_u)
  sq = lambda a: a.reshape(128, 128)
  outs = pl.pallas_call(
      _tc_body,
      out_shape=[jax.ShapeDtypeStruct((1, 1), f32)] * 6,
      out_specs=[pl.BlockSpec(memory_space=pltpu.SMEM)] * 6,
  )(sq(pif), sq(pis), sq(pjf), sq(pjs), sq(rel),
    regp.reshape(4, 128), discp.reshape(4, 128))
  click, l_int, l_p1, l_p2, l_disc, l_reg = [o[0, 0] for o in outs]
  return (click, l_int, l_p1, l_p2, l_disc, l_reg)
